# baseline (speedup 1.0000x reference)
import jax
import jax.numpy as jnp
from jax import lax
from jax.experimental import pallas as pl
from jax.experimental.pallas import tpu as pltpu
from jax.experimental.pallas import tpu_sc as plsc

T = 256
E = 64
TOP_K = 8
DIM = 1024
INTER = 256
S_INTER = 512
E_BLK = 4

NC = 2
NS = 16
LANES = 16
N_CHUNK = T // LANES


def _dot_t(a, b):
    return lax.dot_general(a, b, (((1,), (1,)), ((), ())),
                           preferred_element_type=jnp.float32)


def _gate_body(x_ref, gw_ref, s_ref):
    s_ref[...] = jax.nn.sigmoid(_dot_t(x_ref[...], gw_ref[...]))


def _gate(x, gate_w):
    return pl.pallas_call(
        _gate_body,
        out_shape=jax.ShapeDtypeStruct((T, E), jnp.float32),
    )(x, gate_w)


def _route_body(s_hbm, wd_hbm, s_vmem, st_vmem, m_vmem, wd_vmem):
    wid = lax.axis_index("s") * NC + lax.axis_index("c")

    @pl.when(wid < N_CHUNK)
    def _work():
        base = wid * LANES
        pltpu.sync_copy(s_hbm.at[pl.ds(base, LANES)], s_vmem)
        lane = lax.broadcasted_iota(jnp.int32, (LANES,), 0)
        for e in range(E):
            col = jnp.full((LANES,), e, jnp.int32)
            v = plsc.load_gather(s_vmem, [lane, col])
            st_vmem[e, :] = v
            m_vmem[e, :] = v
        ssum = jnp.zeros((LANES,), jnp.float32)
        for _ in range(TOP_K):
            mx = m_vmem[0, :]
            for e in range(1, E):
                mx = jnp.maximum(mx, m_vmem[e, :])
            ssum = ssum + mx
            found = jnp.zeros((LANES,), jnp.int32)
            for e in range(E):
                v = m_vmem[e, :]
                eq = (v == mx) & (found == 0)
                m_vmem[e, :] = jnp.where(eq, -1.0, v)
                found = found + jnp.where(eq, 1, 0)
        inv = 1.0 / ssum
        for e in range(E):
            col = jnp.full((LANES,), e, jnp.int32)
            sel = m_vmem[e, :] < 0.0
            wdv = jnp.where(sel, st_vmem[e, :] * inv, 0.0)
            plsc.store_scatter(wd_vmem, [lane, col], wdv)
        pltpu.sync_copy(wd_vmem, wd_hbm.at[pl.ds(base, LANES)])


_route = pl.kernel(
    _route_body,
    out_type=jax.ShapeDtypeStruct((T, E), jnp.float32),
    mesh=plsc.VectorSubcoreMesh(core_axis_name="c", subcore_axis_name="s"),
    compiler_params=pltpu.CompilerParams(needs_layout_passes=False),
    scratch_types=[
        pltpu.VMEM((LANES, E), jnp.float32),
        pltpu.VMEM((E, LANES), jnp.float32),
        pltpu.VMEM((E, LANES), jnp.float32),
        pltpu.VMEM((LANES, E), jnp.float32),
    ],
)


def _moe_body(x_ref, wd_ref, w1_ref, w2_ref, w3_ref, sw1_ref, sw2_ref,
              sw3_ref, o_ref):
    i = pl.program_id(0)
    xf = x_ref[...]
    wdt = wd_ref[...]
    eiota = lax.broadcasted_iota(jnp.int32, (E, 1), 0)
    e0 = i * E_BLK
    acc = None
    for j in range(E_BLK):
        h1 = _dot_t(xf, w1_ref[j])
        h3 = _dot_t(xf, w3_ref[j])
        h = h1 * jax.nn.sigmoid(h1) * h3
        yc = _dot_t(h, w2_ref[j])
        onehot = (eiota == (e0 + j)).astype(jnp.float32)
        wcol = jnp.dot(wdt, onehot, preferred_element_type=jnp.float32)
        contrib = yc * wcol
        acc = contrib if acc is None else acc + contrib

    @pl.when(i == 0)
    def _with_shared():
        s1 = _dot_t(xf, sw1_ref[...])
        s3 = _dot_t(xf, sw3_ref[...])
        hs = s1 * jax.nn.sigmoid(s1) * s3
        o_ref[...] = _dot_t(hs, sw2_ref[...]) + acc

    @pl.when(i > 0)
    def _accum():
        o_ref[...] += acc


def _moe(x, wd, W1, W2, W3, SW1, SW2, SW3):
    grid = (E // E_BLK,)
    const = lambda shape: pl.BlockSpec(shape, lambda i: (0,) * len(shape))
    ew = lambda shape: pl.BlockSpec(shape, lambda i: (i, 0, 0))
    return pl.pallas_call(
        _moe_body,
        grid=grid,
        in_specs=[
            const((T, DIM)),
            const((T, E)),
            ew((E_BLK, INTER, DIM)),
            ew((E_BLK, DIM, INTER)),
            ew((E_BLK, INTER, DIM)),
            const((S_INTER, DIM)),
            const((DIM, S_INTER)),
            const((S_INTER, DIM)),
        ],
        out_specs=const((T, DIM)),
        out_shape=jax.ShapeDtypeStruct((T, DIM), jnp.float32),
    )(x, wd, W1, W2, W3, SW1, SW2, SW3)


def kernel(x, gate_w, W1, W2, W3, SW1, SW2, SW3):
    scores_t = _gate(x, gate_w)
    wd = _route(scores_t)
    return _moe(x, wd, W1, W2, W3, SW1, SW2, SW3)


# speedup vs baseline: 1.0379x; 1.0379x over previous
import jax
import jax.numpy as jnp
from jax import lax
from jax.experimental import pallas as pl
from jax.experimental.pallas import tpu as pltpu
from jax.experimental.pallas import tpu_sc as plsc

T = 256
E = 64
TOP_K = 8
DIM = 1024
INTER = 256
S_INTER = 512
E_BLK = 4

NC = 2
NS = 16
LANES = 16
N_CHUNK = T // LANES


def _dot_t(a, b):
    return lax.dot_general(a, b, (((1,), (1,)), ((), ())),
                           preferred_element_type=jnp.float32)


def _gate_body(x_ref, gw_ref, s_ref):
    s_ref[...] = jax.nn.sigmoid(_dot_t(x_ref[...], gw_ref[...]))


def _gate(x, gate_w):
    return pl.pallas_call(
        _gate_body,
        out_shape=jax.ShapeDtypeStruct((T, E), jnp.float32),
    )(x, gate_w)


def _route_body(s_hbm, wd_hbm, s_vmem, st_vmem, m_vmem, wd_vmem):
    wid = lax.axis_index("s") * NC + lax.axis_index("c")

    @pl.when(wid < N_CHUNK)
    def _work():
        base = wid * LANES
        pltpu.sync_copy(s_hbm.at[pl.ds(base, LANES)], s_vmem)
        lane = lax.broadcasted_iota(jnp.int32, (LANES,), 0)
        for e in range(E):
            col = jnp.full((LANES,), e, jnp.int32)
            v = plsc.load_gather(s_vmem, [lane, col])
            st_vmem[e, :] = v
            m_vmem[e, :] = v
        ssum = jnp.zeros((LANES,), jnp.float32)
        for _ in range(TOP_K):
            mx = m_vmem[0, :]
            for e in range(1, E):
                mx = jnp.maximum(mx, m_vmem[e, :])
            ssum = ssum + mx
            found = jnp.zeros((LANES,), jnp.int32)
            for e in range(E):
                v = m_vmem[e, :]
                eq = (v == mx) & (found == 0)
                m_vmem[e, :] = jnp.where(eq, -1.0, v)
                found = found + jnp.where(eq, 1, 0)
        inv = 1.0 / ssum
        for e in range(E):
            col = jnp.full((LANES,), e, jnp.int32)
            sel = m_vmem[e, :] < 0.0
            wdv = jnp.where(sel, st_vmem[e, :] * inv, 0.0)
            plsc.store_scatter(wd_vmem, [lane, col], wdv)
        pltpu.sync_copy(wd_vmem, wd_hbm.at[pl.ds(base, LANES)])


_route = pl.kernel(
    _route_body,
    out_type=jax.ShapeDtypeStruct((T, E), jnp.float32),
    mesh=plsc.VectorSubcoreMesh(core_axis_name="c", subcore_axis_name="s"),
    compiler_params=pltpu.CompilerParams(needs_layout_passes=False),
    scratch_types=[
        pltpu.VMEM((LANES, E), jnp.float32),
        pltpu.VMEM((E, LANES), jnp.float32),
        pltpu.VMEM((E, LANES), jnp.float32),
        pltpu.VMEM((LANES, E), jnp.float32),
    ],
)


def _shared_body(x_ref, sw1_ref, sw2_ref, sw3_ref, z_ref):
    xf = x_ref[...]
    s1 = _dot_t(xf, sw1_ref[...])
    s3 = _dot_t(xf, sw3_ref[...])
    hs = s1 * jax.nn.sigmoid(s1) * s3
    z_ref[...] = _dot_t(hs, sw2_ref[...])


def _shared(x, SW1, SW2, SW3):
    return pl.pallas_call(
        _shared_body,
        out_shape=jax.ShapeDtypeStruct((T, DIM), jnp.float32),
    )(x, SW1, SW2, SW3)


def _moe_body(x_ref, wd_ref, z_ref, w1_ref, w2_ref, w3_ref, o_ref):
    i = pl.program_id(0)
    xf = x_ref[...]
    wdt = wd_ref[...]
    eiota = lax.broadcasted_iota(jnp.int32, (E, 1), 0)
    e0 = i * E_BLK
    acc = None
    for j in range(E_BLK):
        h1 = _dot_t(xf, w1_ref[j])
        h3 = _dot_t(xf, w3_ref[j])
        h = h1 * jax.nn.sigmoid(h1) * h3
        yc = _dot_t(h, w2_ref[j])
        onehot = (eiota == (e0 + j)).astype(jnp.float32)
        wcol = jnp.dot(wdt, onehot, preferred_element_type=jnp.float32)
        contrib = yc * wcol
        acc = contrib if acc is None else acc + contrib

    @pl.when(i == 0)
    def _with_shared():
        o_ref[...] = z_ref[...] + acc

    @pl.when(i > 0)
    def _accum():
        o_ref[...] += acc


def _moe(x, wd, z, W1, W2, W3):
    grid = (E // E_BLK,)
    const = lambda shape: pl.BlockSpec(shape, lambda i: (0,) * len(shape))
    ew = lambda shape: pl.BlockSpec(shape, lambda i: (i, 0, 0))
    return pl.pallas_call(
        _moe_body,
        grid=grid,
        in_specs=[
            const((T, DIM)),
            const((T, E)),
            const((T, DIM)),
            ew((E_BLK, INTER, DIM)),
            ew((E_BLK, DIM, INTER)),
            ew((E_BLK, INTER, DIM)),
        ],
        out_specs=const((T, DIM)),
        out_shape=jax.ShapeDtypeStruct((T, DIM), jnp.float32),
    )(x, wd, z, W1, W2, W3)


def kernel(x, gate_w, W1, W2, W3, SW1, SW2, SW3):
    scores = _gate(x, gate_w)
    wd = _route(scores)
    z = _shared(x, SW1, SW2, SW3)
    return _moe(x, wd, z, W1, W2, W3)
